# Initial kernel scaffold; baseline (speedup 1.0000x reference)
import jax
import jax.numpy as jnp
from jax.experimental import pallas as pl


def kernel(x_seq, edge_index, edge_attr, tw1, tb1, inw1, inb1, gw1, gb1, tw2, tb2, inw2, inb2, gw2, gb2, ow, ob):
    raise NotImplementedError("write your pallas kernel here")


# speedup vs baseline: 2.2799x; 2.2799x over previous
import functools
import jax
import jax.numpy as jnp
from jax import lax
from jax.experimental import pallas as pl
from jax.experimental.pallas import tpu as pltpu

_N = 10000
_T = 12
_C = 128
_NB = 200


def _dinv_from_deg(deg_ref, nb):
    deg = 1.0 + deg_ref[0, :, 0:1] + deg_ref[1, :, 0:1]
    return lax.rsqrt(deg)


def _temporal(xs, w0, w1, w2, tb, inw, inb):
    T = len(xs)
    ys = []
    for t in range(T):
        acc = jnp.dot(xs[t], w1, preferred_element_type=jnp.float32)
        if t > 0:
            acc = acc + jnp.dot(xs[t - 1], w0, preferred_element_type=jnp.float32)
        if t < T - 1:
            acc = acc + jnp.dot(xs[t + 1], w2, preferred_element_type=jnp.float32)
        ys.append(acc + tb)
    y = jnp.stack(ys)
    mean = jnp.mean(y, axis=0, keepdims=True)
    var = jnp.mean((y - mean) * (y - mean), axis=0, keepdims=True)
    yn = (y - mean) * lax.rsqrt(var + 1e-5) * inw + inb
    yr = jnp.maximum(yn, 0.0)
    return [yr[t] for t in range(T)]


def _tc_layer1_body(x_ref, deg_ref, w0_ref, w1_ref, w2_ref, tb_ref, inw_ref,
                    inb_ref, gw_ref, xw_ref, xs_ref):
    x = x_ref[...]
    yr = _temporal([x[t] for t in range(_T)], w0_ref[...], w1_ref[...],
                   w2_ref[...], tb_ref[...], inw_ref[...], inb_ref[...])
    dinv = _dinv_from_deg(deg_ref, _NB)
    gw = gw_ref[...]
    for t in range(_T):
        xw = jnp.dot(yr[t], gw, preferred_element_type=jnp.float32)
        xw_ref[t] = xw
        xs_ref[t] = xw * dinv


def _tc_layer2_body(s_ref, xwp_ref, deg_ref, gb_ref, w0_ref, w1_ref, w2_ref,
                    tb_ref, inw_ref, inb_ref, gw_ref, xw_ref, xs_ref):
    dinv = _dinv_from_deg(deg_ref, _NB)
    dinv2 = dinv * dinv
    gb = gb_ref[...]
    hs = [dinv * s_ref[t] + dinv2 * xwp_ref[t] + gb for t in range(_T)]
    yr = _temporal(hs, w0_ref[...], w1_ref[...], w2_ref[...], tb_ref[...],
                   inw_ref[...], inb_ref[...])
    dinv_ = dinv
    gw = gw_ref[...]
    for t in range(_T):
        xw = jnp.dot(yr[t], gw, preferred_element_type=jnp.float32)
        xw_ref[t] = xw
        xs_ref[t] = xw * dinv_


def _tc_out_body(s_ref, xwp_ref, deg_ref, gb_ref, ow_ref, ob_ref, out_ref):
    dinv = _dinv_from_deg(deg_ref, _NB)
    dinv2 = dinv * dinv
    gb = gb_ref[...]
    acc = jnp.zeros((_NB, _C), jnp.float32)
    for t in range(_T):
        acc = acc + (dinv * s_ref[t] + dinv2 * xwp_ref[t] + gb)
    hm = acc * (1.0 / _T)
    out_ref[...] = jnp.dot(hm, ow_ref[...], preferred_element_type=jnp.float32) + ob_ref[...]


def _full_spec(shape):
    nd = len(shape)
    return pl.BlockSpec(shape, lambda i, _nd=nd: (0,) * _nd)


def _node_spec3(d0, d2):
    return pl.BlockSpec((d0, _NB, d2), lambda i: (0, i, 0))


_GRID = _N // _NB


def _run_layer1(x_seq, deg16, w0, w1, w2, tb, inw, inb, gw):
    return pl.pallas_call(
        _tc_layer1_body,
        grid=(_GRID,),
        in_specs=[
            _node_spec3(_T, _C),
            _node_spec3(2, 16),
            _full_spec((_C, _C)), _full_spec((_C, _C)), _full_spec((_C, _C)),
            _full_spec((1, _C)), _full_spec((1, _C)), _full_spec((1, _C)),
            _full_spec((_C, _C)),
        ],
        out_specs=[_node_spec3(_T, _C), _node_spec3(_T, _C)],
        out_shape=[jax.ShapeDtypeStruct((_T, _N, _C), jnp.float32)] * 2,
    )(x_seq, deg16, w0, w1, w2, tb, inw, inb, gw)


def _run_layer2(s1, xw1, deg16, gb, w0, w1, w2, tb, inw, inb, gw):
    return pl.pallas_call(
        _tc_layer2_body,
        grid=(_GRID,),
        in_specs=[
            _node_spec3(_T, _C), _node_spec3(_T, _C), _node_spec3(2, 16),
            _full_spec((1, _C)),
            _full_spec((_C, _C)), _full_spec((_C, _C)), _full_spec((_C, _C)),
            _full_spec((1, _C)), _full_spec((1, _C)), _full_spec((1, _C)),
            _full_spec((_C, _C)),
        ],
        out_specs=[_node_spec3(_T, _C), _node_spec3(_T, _C)],
        out_shape=[jax.ShapeDtypeStruct((_T, _N, _C), jnp.float32)] * 2,
    )(s1, xw1, deg16, gb, w0, w1, w2, tb, inw, inb, gw)


def _run_out(s2, xw2, deg16, gb, ow, ob):
    return pl.pallas_call(
        _tc_out_body,
        grid=(_GRID,),
        in_specs=[
            _node_spec3(_T, _C), _node_spec3(_T, _C), _node_spec3(2, 16),
            _full_spec((1, _C)), _full_spec((_C, _C)), _full_spec((1, _C)),
        ],
        out_specs=pl.BlockSpec((_NB, _C), lambda i: (i, 0)),
        out_shape=jax.ShapeDtypeStruct((_N, _C), jnp.float32),
    )(s2, xw2, deg16, gb, ow, ob)


def _agg(xs, src, dst, ew):
    msgs = xs[:, src, :] * ew[None, :, None]
    msgs = msgs.transpose(1, 0, 2).reshape(src.shape[0], _T * _C)
    s = jax.ops.segment_sum(msgs, dst, num_segments=_N)
    return s.reshape(_N, _T, _C).transpose(1, 0, 2)


def _deg_partials(dst, ew):
    d = jax.ops.segment_sum(ew, dst, num_segments=_N)
    deg16 = jnp.zeros((2, _N, 16), jnp.float32)
    return deg16.at[0, :, 0].set(d)


def kernel(x_seq, edge_index, edge_attr, tw1, tb1, inw1, inb1, gw1, gb1,
           tw2, tb2, inw2, inb2, gw2, gb2, ow, ob):
    src = edge_index[0]
    dst = edge_index[1]
    ew = edge_attr[:, 0]

    def prep(tw):
        return tw[:, :, 0].T, tw[:, :, 1].T, tw[:, :, 2].T

    w10, w11, w12 = prep(tw1)
    w20, w21, w22 = prep(tw2)
    r = lambda v: v.reshape(1, _C)

    deg16 = _deg_partials(dst, ew)

    xw1, xs1 = _run_layer1(x_seq, deg16, w10, w11, w12, r(tb1), r(inw1),
                           r(inb1), gw1)
    s1 = _agg(xs1, src, dst, ew)
    xw2, xs2 = _run_layer2(s1, xw1, deg16, r(gb1), w20, w21, w22, r(tb2),
                           r(inw2), r(inb2), gw2)
    s2 = _agg(xs2, src, dst, ew)
    return _run_out(s2, xw2, deg16, r(gb2), ow, r(ob))


# speedup vs baseline: 3.2206x; 1.4126x over previous
import functools
import jax
import jax.numpy as jnp
from jax import lax
from jax.experimental import pallas as pl
from jax.experimental.pallas import tpu as pltpu
from jax.experimental.pallas import tpu_sc as plsc

_N = 10000
_T = 12
_C = 128
_NB = 200

_EB = 64
_EPAD = 163840
_NCH = _EPAD // _EB
_CH_W = _NCH // 16
_NPAD = 10240
_NROW = _NPAD // 16
_ZB = 16
_TH = _T // 2


def _sc_agg_kernel(xs_hbm, srcf_hbm, dst_hbm, ew_hbm, out_hbm,
                   idx_v, dst_v, ew_v, rows_v, zbuf, shared, sem):
    cid = lax.axis_index("c")
    sid = lax.axis_index("s")
    zero16 = jnp.zeros((16,), jnp.float32)
    for i in range(_ZB):
        for h in range(_C // 16):
            zbuf[i, pl.ds(h * 16, 16)] = zero16

    def one_t(tt, carry):
        t = cid * _TH + tt
        for k in range(_NROW // _ZB):
            pltpu.sync_copy(zbuf, shared.at[pl.ds(sid * _NROW + k * _ZB, _ZB)])
        plsc.subcore_barrier()

        def one_chunk(k, c2):
            c = sid * _CH_W + k
            pltpu.sync_copy(srcf_hbm.at[pl.ds(t * _EPAD + c * _EB, _EB)], idx_v)
            pltpu.sync_copy(dst_hbm.at[pl.ds(c * _EB, _EB)], dst_v)
            pltpu.sync_copy(ew_hbm.at[c], ew_v)
            pltpu.async_copy(xs_hbm.at[idx_v], rows_v, sem).wait()
            for e in range(_EB):
                w = ew_v[e, :]
                for h in range(_C // 16):
                    sl = pl.ds(h * 16, 16)
                    rows_v[e, sl] = rows_v[e, sl] * w
            pltpu.sync_copy(rows_v, shared.at[dst_v], add=True)
            return c2

        lax.fori_loop(0, _CH_W, one_chunk, 0)
        plsc.subcore_barrier()
        pltpu.sync_copy(shared.at[pl.ds(sid * _NROW, _NROW)],
                        out_hbm.at[t, pl.ds(sid * _NROW, _NROW)])
        plsc.subcore_barrier()
        return carry

    lax.fori_loop(0, _TH, one_t, 0)


@jax.jit
def _sc_agg(xs_flat, srcf, dst3, ew16):
    f = functools.partial(
        pl.kernel,
        mesh=plsc.VectorSubcoreMesh(core_axis_name="c", subcore_axis_name="s"),
        out_type=jax.ShapeDtypeStruct((_T, _NPAD, _C), jnp.float32),
        scratch_types=[
            pltpu.VMEM((_EB,), jnp.int32),
            pltpu.VMEM((_EB,), jnp.int32),
            pltpu.VMEM((_EB, 16), jnp.float32),
            pltpu.VMEM((_EB, _C), jnp.float32),
            pltpu.VMEM((_ZB, _C), jnp.float32),
            pltpu.VMEM_SHARED((_NPAD, _C), jnp.float32),
            pltpu.SemaphoreType.DMA,
        ],
    )(_sc_agg_kernel)
    return f(xs_flat, srcf, dst3, ew16)


def _dinv_from_deg(deg_ref, nb):
    deg = 1.0 + deg_ref[0, :, 0:1] + deg_ref[1, :, 0:1]
    return lax.rsqrt(deg)


def _temporal(xs, w0, w1, w2, tb, inw, inb):
    T = len(xs)
    ys = []
    for t in range(T):
        acc = jnp.dot(xs[t], w1, preferred_element_type=jnp.float32)
        if t > 0:
            acc = acc + jnp.dot(xs[t - 1], w0, preferred_element_type=jnp.float32)
        if t < T - 1:
            acc = acc + jnp.dot(xs[t + 1], w2, preferred_element_type=jnp.float32)
        ys.append(acc + tb)
    y = jnp.stack(ys)
    mean = jnp.mean(y, axis=0, keepdims=True)
    var = jnp.mean((y - mean) * (y - mean), axis=0, keepdims=True)
    yn = (y - mean) * lax.rsqrt(var + 1e-5) * inw + inb
    yr = jnp.maximum(yn, 0.0)
    return [yr[t] for t in range(T)]


def _tc_layer1_body(x_ref, deg_ref, w0_ref, w1_ref, w2_ref, tb_ref, inw_ref,
                    inb_ref, gw_ref, xw_ref, xs_ref):
    x = x_ref[...]
    yr = _temporal([x[t] for t in range(_T)], w0_ref[...], w1_ref[...],
                   w2_ref[...], tb_ref[...], inw_ref[...], inb_ref[...])
    dinv = _dinv_from_deg(deg_ref, _NB)
    gw = gw_ref[...]
    for t in range(_T):
        xw = jnp.dot(yr[t], gw, preferred_element_type=jnp.float32)
        xw_ref[t] = xw
        xs_ref[t] = xw * dinv


def _tc_layer2_body(s_ref, xwp_ref, deg_ref, gb_ref, w0_ref, w1_ref, w2_ref,
                    tb_ref, inw_ref, inb_ref, gw_ref, xw_ref, xs_ref):
    dinv = _dinv_from_deg(deg_ref, _NB)
    dinv2 = dinv * dinv
    gb = gb_ref[...]
    hs = [dinv * s_ref[t] + dinv2 * xwp_ref[t] + gb for t in range(_T)]
    yr = _temporal(hs, w0_ref[...], w1_ref[...], w2_ref[...], tb_ref[...],
                   inw_ref[...], inb_ref[...])
    dinv_ = dinv
    gw = gw_ref[...]
    for t in range(_T):
        xw = jnp.dot(yr[t], gw, preferred_element_type=jnp.float32)
        xw_ref[t] = xw
        xs_ref[t] = xw * dinv_


def _tc_out_body(s_ref, xwp_ref, deg_ref, gb_ref, ow_ref, ob_ref, out_ref):
    dinv = _dinv_from_deg(deg_ref, _NB)
    dinv2 = dinv * dinv
    gb = gb_ref[...]
    acc = jnp.zeros((_NB, _C), jnp.float32)
    for t in range(_T):
        acc = acc + (dinv * s_ref[t] + dinv2 * xwp_ref[t] + gb)
    hm = acc * (1.0 / _T)
    out_ref[...] = jnp.dot(hm, ow_ref[...], preferred_element_type=jnp.float32) + ob_ref[...]


def _full_spec(shape):
    nd = len(shape)
    return pl.BlockSpec(shape, lambda i, _nd=nd: (0,) * _nd)


def _node_spec3(d0, d2):
    return pl.BlockSpec((d0, _NB, d2), lambda i: (0, i, 0))


_GRID = _N // _NB


def _run_layer1(x_seq, deg16, w0, w1, w2, tb, inw, inb, gw):
    return pl.pallas_call(
        _tc_layer1_body,
        grid=(_GRID,),
        in_specs=[
            _node_spec3(_T, _C),
            _node_spec3(2, 16),
            _full_spec((_C, _C)), _full_spec((_C, _C)), _full_spec((_C, _C)),
            _full_spec((1, _C)), _full_spec((1, _C)), _full_spec((1, _C)),
            _full_spec((_C, _C)),
        ],
        out_specs=[_node_spec3(_T, _C), _node_spec3(_T, _C)],
        out_shape=[jax.ShapeDtypeStruct((_T, _N, _C), jnp.float32)] * 2,
    )(x_seq, deg16, w0, w1, w2, tb, inw, inb, gw)


def _run_layer2(s1, xw1, deg16, gb, w0, w1, w2, tb, inw, inb, gw):
    return pl.pallas_call(
        _tc_layer2_body,
        grid=(_GRID,),
        in_specs=[
            _node_spec3(_T, _C), _node_spec3(_T, _C), _node_spec3(2, 16),
            _full_spec((1, _C)),
            _full_spec((_C, _C)), _full_spec((_C, _C)), _full_spec((_C, _C)),
            _full_spec((1, _C)), _full_spec((1, _C)), _full_spec((1, _C)),
            _full_spec((_C, _C)),
        ],
        out_specs=[_node_spec3(_T, _C), _node_spec3(_T, _C)],
        out_shape=[jax.ShapeDtypeStruct((_T, _N, _C), jnp.float32)] * 2,
    )(s1, xw1, deg16, gb, w0, w1, w2, tb, inw, inb, gw)


def _run_out(s2, xw2, deg16, gb, ow, ob):
    return pl.pallas_call(
        _tc_out_body,
        grid=(_GRID,),
        in_specs=[
            _node_spec3(_T, _C), _node_spec3(_T, _C), _node_spec3(2, 16),
            _full_spec((1, _C)), _full_spec((_C, _C)), _full_spec((1, _C)),
        ],
        out_specs=pl.BlockSpec((_NB, _C), lambda i: (i, 0)),
        out_shape=jax.ShapeDtypeStruct((_N, _C), jnp.float32),
    )(s2, xw2, deg16, gb, ow, ob)


def _edge_tables(src, dst, ew):
    e = src.shape[0]
    pad = _EPAD - e
    src_p = jnp.concatenate([src, jnp.zeros((pad,), src.dtype)])
    dst_p = jnp.concatenate([dst, jnp.zeros((pad,), dst.dtype)])
    ew_p = jnp.concatenate([ew, jnp.zeros((pad,), ew.dtype)])
    t_off = (jnp.arange(_T, dtype=jnp.int32) * _N)[:, None]
    srcf = (src_p[None, :] + t_off).reshape(_T * _EPAD)
    ew16 = jnp.broadcast_to(ew_p[:, None], (_EPAD, 16)).reshape(_NCH, _EB, 16)
    return srcf, dst_p, ew16


def _agg(xs, srcf, dst3, ew16):
    s = _sc_agg(xs.reshape(_T * _N, _C), srcf, dst3, ew16)
    return s[:, :_N, :]


def _deg_partials(dst, ew):
    d = jax.ops.segment_sum(ew, dst, num_segments=_N)
    deg16 = jnp.zeros((2, _N, 16), jnp.float32)
    return deg16.at[0, :, 0].set(d)


def kernel(x_seq, edge_index, edge_attr, tw1, tb1, inw1, inb1, gw1, gb1,
           tw2, tb2, inw2, inb2, gw2, gb2, ow, ob):
    src = edge_index[0]
    dst = edge_index[1]
    ew = edge_attr[:, 0]

    def prep(tw):
        return tw[:, :, 0].T, tw[:, :, 1].T, tw[:, :, 2].T

    w10, w11, w12 = prep(tw1)
    w20, w21, w22 = prep(tw2)
    r = lambda v: v.reshape(1, _C)

    deg16 = _deg_partials(dst, ew)
    srcf, dst3, ew16 = _edge_tables(src, dst, ew)

    xw1, xs1 = _run_layer1(x_seq, deg16, w10, w11, w12, r(tb1), r(inw1),
                           r(inb1), gw1)
    s1 = _agg(xs1, srcf, dst3, ew16)
    xw2, xs2 = _run_layer2(s1, xw1, deg16, r(gb1), w20, w21, w22, r(tb2),
                           r(inw2), r(inb2), gw2)
    s2 = _agg(xs2, srcf, dst3, ew16)
    return _run_out(s2, xw2, deg16, r(gb2), ow, r(ob))
